# baseline (speedup 1.0000x reference)
import functools

import jax
import jax.numpy as jnp
from jax import lax
from jax.experimental import pallas as pl
from jax.experimental.pallas import tpu as pltpu
from jax.experimental.pallas import tpu_sc as plsc

_NC = 2
_NS = 16
_NW = _NC * _NS
_L = 16


def _sc_add(inp_flat, tab_flat, B, S, D):
    seq_per_w = S // _NW
    C = 32
    n_chunks = seq_per_w // C
    CHUNK = C * D

    mesh = plsc.VectorSubcoreMesh(core_axis_name="c", subcore_axis_name="s")

    @functools.partial(
        pl.kernel,
        out_type=jax.ShapeDtypeStruct((B * S * D,), jnp.float32),
        mesh=mesh,
        scratch_types=[
            pltpu.VMEM((CHUNK,), jnp.float32),
            pltpu.VMEM((CHUNK,), jnp.float32),
        ],
    )
    def body(inp_hbm, tab_hbm, out_hbm, tbuf, dbuf):
        w = lax.axis_index("s") * _NC + lax.axis_index("c")

        def chunk_body(c, carry):
            seq0 = (w * seq_per_w + c * C) * D
            pltpu.sync_copy(tab_hbm.at[pl.ds(seq0, CHUNK)], tbuf)

            def batch_body(b, carry2):
                off = b * (S * D) + seq0
                pltpu.sync_copy(inp_hbm.at[pl.ds(off, CHUNK)], dbuf)

                @plsc.parallel_loop(0, CHUNK, _L, unroll=8)
                def add_body(i):
                    sl = pl.ds(i, _L)
                    dbuf[sl] = dbuf[sl] + tbuf[sl]

                pltpu.sync_copy(dbuf, out_hbm.at[pl.ds(off, CHUNK)])
                return carry2

            return lax.fori_loop(0, B, batch_body, carry)

        lax.fori_loop(0, n_chunks, chunk_body, 0)

    return body(inp_flat, tab_flat)


def kernel(inp, embed_table):
    B, S, D = inp.shape
    out_flat = _sc_add(
        inp.reshape(B * S * D), embed_table[:S].reshape(S * D), B, S, D
    )
    return out_flat.reshape(B, S, D)


# speedup vs baseline: 1.2485x; 1.2485x over previous
import jax
import jax.numpy as jnp
from jax import lax
from jax.experimental import pallas as pl
from jax.experimental.pallas import tpu as pltpu
from jax.experimental.pallas import tpu_sc as plsc

_NC = 2
_NS = 16
_NW = _NC * _NS
_L = 16


def _sc_add(inp_flat, tab_flat, B, S, D):
    seq_per_w = S // _NW
    C = 8
    n_chunks = seq_per_w // C
    n_pairs = n_chunks // 2
    CHUNK = C * D

    mesh = plsc.VectorSubcoreMesh(core_axis_name="c", subcore_axis_name="s")

    @pl.kernel(
        out_type=jax.ShapeDtypeStruct((B * S * D,), jnp.float32),
        mesh=mesh,
        scratch_types=[
            [pltpu.VMEM((CHUNK,), jnp.float32) for _ in range(2)],
            [[pltpu.VMEM((CHUNK,), jnp.float32) for _ in range(2)]
             for _ in range(B)],
            pltpu.SemaphoreType.DMA((2,)),
            pltpu.SemaphoreType.DMA((B, 2)),
            pltpu.SemaphoreType.DMA((B, 2)),
        ],
    )
    def body(inp_hbm, tab_hbm, out_hbm, tbufs, dbufs, tsems, lsems, ssems):
        w = lax.axis_index("s") * _NC + lax.axis_index("c")
        base = w * seq_per_w * D

        def tab_slice(c):
            return tab_hbm.at[pl.ds(base + c * CHUNK, CHUNK)]

        def in_slice(c, b):
            return inp_hbm.at[pl.ds(b * (S * D) + base + c * CHUNK, CHUNK)]

        def out_slice(c, b):
            return out_hbm.at[pl.ds(b * (S * D) + base + c * CHUNK, CHUNK)]

        pltpu.async_copy(tab_slice(0), tbufs[0], tsems.at[0])
        for b in range(B):
            pltpu.async_copy(in_slice(0, b), dbufs[b][0], lsems.at[b, 0])

        def step(c2, par):
            c = c2 * 2 + par
            nxt = 1 - par

            def tab_prefetch():
                pltpu.async_copy(tab_slice(c + 1), tbufs[nxt], tsems.at[nxt])

            if par == 0:
                tab_prefetch()
            else:
                pl.when(c2 < n_pairs - 1)(tab_prefetch)

            for b in range(B):
                def store_wait(b=b):
                    pltpu.make_async_copy(
                        dbufs[b][nxt], out_slice(c - 1, b), ssems.at[b, nxt]
                    ).wait()

                def inp_prefetch(b=b):
                    pltpu.async_copy(
                        in_slice(c + 1, b), dbufs[b][nxt], lsems.at[b, nxt]
                    )

                if par == 0:
                    pl.when(c2 > 0)(store_wait)
                    inp_prefetch()
                else:
                    store_wait()
                    pl.when(c2 < n_pairs - 1)(inp_prefetch)

            pltpu.make_async_copy(tab_slice(c), tbufs[par], tsems.at[par]).wait()
            for b in range(B):
                pltpu.make_async_copy(
                    in_slice(c, b), dbufs[b][par], lsems.at[b, par]
                ).wait()

            tbuf = tbufs[par]
            cur = [dbufs[b][par] for b in range(B)]

            @plsc.parallel_loop(0, CHUNK, _L, unroll=4)
            def add_body(i):
                sl = pl.ds(i, _L)
                t = tbuf[sl]
                for b in range(B):
                    cur[b][sl] = cur[b][sl] + t

            for b in range(B):
                pltpu.async_copy(
                    dbufs[b][par], out_slice(c, b), ssems.at[b, par]
                )

        def pair_body(c2, carry):
            step(c2, 0)
            step(c2, 1)
            return carry

        lax.fori_loop(0, n_pairs, pair_body, 0)

        for b in range(B):
            pltpu.make_async_copy(
                dbufs[b][1], out_slice(n_chunks - 1, b), ssems.at[b, 1]
            ).wait()

    return body(inp_flat, tab_flat)


def kernel(inp, embed_table):
    B, S, D = inp.shape
    out_flat = _sc_add(
        inp.reshape(B * S * D), embed_table[:S].reshape(S * D), B, S, D
    )
    return out_flat.reshape(B, S, D)


# speedup vs baseline: 3.7632x; 3.0142x over previous
import jax
import jax.numpy as jnp
from jax import lax
from jax.experimental import pallas as pl
from jax.experimental.pallas import tpu as pltpu
from jax.experimental.pallas import tpu_sc as plsc

_NC = 2
_NS = 16
_NW = _NC * _NS
_L = 16


def _sc_add(inp, tab, B, S, D):
    seq_per_w = S // _NW
    C = 8
    n_chunks = seq_per_w // C
    n_pairs = n_chunks // 2

    mesh = plsc.VectorSubcoreMesh(core_axis_name="c", subcore_axis_name="s")

    @pl.kernel(
        out_type=jax.ShapeDtypeStruct((B, S, D), jnp.float32),
        mesh=mesh,
        scratch_types=[
            [pltpu.VMEM((C, D), jnp.float32) for _ in range(2)],
            [[pltpu.VMEM((C, D), jnp.float32) for _ in range(2)]
             for _ in range(B)],
            pltpu.SemaphoreType.DMA((2,)),
            pltpu.SemaphoreType.DMA((B, 2)),
            pltpu.SemaphoreType.DMA((B, 2)),
        ],
        compiler_params=pltpu.CompilerParams(use_tc_tiling_on_sc=True),
    )
    def body(inp_hbm, tab_hbm, out_hbm, tbufs, dbufs, tsems, lsems, ssems):
        w = lax.axis_index("s") * _NC + lax.axis_index("c")
        base = w * seq_per_w

        def tab_slice(c):
            return tab_hbm.at[pl.ds(base + c * C, C), :]

        def in_slice(c, b):
            return inp_hbm.at[b, pl.ds(base + c * C, C), :]

        def out_slice(c, b):
            return out_hbm.at[b, pl.ds(base + c * C, C), :]

        pltpu.async_copy(tab_slice(0), tbufs[0], tsems.at[0])
        for b in range(B):
            pltpu.async_copy(in_slice(0, b), dbufs[b][0], lsems.at[b, 0])

        def step(c2, par):
            c = c2 * 2 + par
            nxt = 1 - par

            def tab_prefetch():
                pltpu.async_copy(tab_slice(c + 1), tbufs[nxt], tsems.at[nxt])

            if par == 0:
                tab_prefetch()
            else:
                pl.when(c2 < n_pairs - 1)(tab_prefetch)

            for b in range(B):
                def store_wait(b=b):
                    pltpu.make_async_copy(
                        dbufs[b][nxt], out_slice(c - 1, b), ssems.at[b, nxt]
                    ).wait()

                def inp_prefetch(b=b):
                    pltpu.async_copy(
                        in_slice(c + 1, b), dbufs[b][nxt], lsems.at[b, nxt]
                    )

                if par == 0:
                    pl.when(c2 > 0)(store_wait)
                    inp_prefetch()
                else:
                    store_wait()
                    pl.when(c2 < n_pairs - 1)(inp_prefetch)

            pltpu.make_async_copy(tab_slice(c), tbufs[par], tsems.at[par]).wait()
            for b in range(B):
                pltpu.make_async_copy(
                    in_slice(c, b), dbufs[b][par], lsems.at[b, par]
                ).wait()

            tbuf = tbufs[par]
            cur = [dbufs[b][par] for b in range(B)]

            for r in range(C):
                @plsc.parallel_loop(0, D, _L, unroll=4)
                def add_body(i):
                    sl = pl.ds(i, _L)
                    t = tbuf[r, sl]
                    for b in range(B):
                        cur[b][r, sl] = cur[b][r, sl] + t

            for b in range(B):
                pltpu.async_copy(
                    dbufs[b][par], out_slice(c, b), ssems.at[b, par]
                )

        def pair_body(c2, carry):
            step(c2, 0)
            step(c2, 1)
            return carry

        lax.fori_loop(0, n_pairs, pair_body, 0)

        for b in range(B):
            pltpu.make_async_copy(
                dbufs[b][1], out_slice(n_chunks - 1, b), ssems.at[b, 1]
            ).wait()

    return body(inp, tab)


def kernel(inp, embed_table):
    B, S, D = inp.shape
    return _sc_add(inp, embed_table[:S], B, S, D)
